# baseline (speedup 1.0000x reference)
import jax
import jax.numpy as jnp
from jax import lax
from jax.experimental import pallas as pl
from jax.experimental.pallas import tpu as pltpu
from jax.experimental.pallas import tpu_sc as plsc

_N = 10000
_E = 320000
_D_FEAT = 128
_HIDDEN = 32
_MLP_H = 64
_D_EDGE = 16

_NC = 2
_NS = 16
_NW = _NC * _NS
_EW = _E // _NW
_CB = 128
_PB = _CB // 4
_NFULL = _EW // _CB
_REM = _EW - _NFULL * _CB
_PREM = _REM // 4
_NSLOT = 3

_BE = 12800
_BP = _BE // 4


def _encode_body(x_ref, wnet_ref, bne_ref, z_ref):
    z_ref[...] = (
        jnp.dot(x_ref[...], wnet_ref[...], preferred_element_type=jnp.float32)
        + bne_ref[...]
    )


def _sc_gather_body(z_ref, ei_ref, ea_ref, gs_ref, gd_ref, ep_ref,
                    idx_s, idx_d, rows_s, rows_d, rows_e,
                    sem_s0, sem_s1, sem_s2, sem_d0, sem_d1, sem_d2,
                    sem_e0, sem_e1, sem_e2):
    wid = lax.axis_index("s") * _NC + lax.axis_index("c")
    base0 = wid * _EW
    pbase0 = base0 // 4
    sems_s = (sem_s0, sem_s1, sem_s2)
    sems_d = (sem_d0, sem_d1, sem_d2)
    sems_e = (sem_e0, sem_e1, sem_e2)

    pltpu.sync_copy(ei_ref.at[pl.ds(base0, _EW)], idx_s)
    pltpu.sync_copy(ei_ref.at[pl.ds(_E + base0, _EW)], idx_d)

    def start_chunk(c, k):
        pltpu.async_copy(z_ref.at[idx_s.at[pl.ds(c * _CB, _CB)]],
                         rows_s.at[k], sems_s[k])
        pltpu.async_copy(z_ref.at[idx_d.at[pl.ds(c * _CB, _CB)]],
                         rows_d.at[k], sems_d[k])
        prow = pbase0 + c * _PB
        pltpu.async_copy(ea_ref.at[pl.ds(prow, _PB)],
                         rows_e.at[k, :, :, pl.ds(0, _D_EDGE)], sems_e[k])
        pltpu.async_copy(ea_ref.at[pl.ds(prow, _PB)],
                         rows_e.at[k, :, :, pl.ds(_D_EDGE, _D_EDGE)], sems_e[k])

    for k in range(_NSLOT):
        start_chunk(k, k)

    def body(i, carry):
        for k in range(_NSLOT):
            c = i * _NSLOT + k
            prow = pbase0 + c * _PB
            pltpu.make_async_copy(z_ref.at[idx_s.at[pl.ds(c * _CB, _CB)]],
                                  rows_s.at[k], sems_s[k]).wait()
            pltpu.make_async_copy(z_ref.at[idx_d.at[pl.ds(c * _CB, _CB)]],
                                  rows_d.at[k], sems_d[k]).wait()
            pltpu.make_async_copy(ea_ref.at[pl.ds(prow, _PB)],
                                  rows_e.at[k, :, :, pl.ds(0, _D_EDGE)],
                                  sems_e[k]).wait()
            pltpu.make_async_copy(ea_ref.at[pl.ds(prow, _PB)],
                                  rows_e.at[k, :, :, pl.ds(_D_EDGE, _D_EDGE)],
                                  sems_e[k]).wait()
            wb_s = pltpu.async_copy(
                rows_s.at[k], gs_ref.at[pl.ds(base0 + c * _CB, _CB)], sems_s[k])
            wb_d = pltpu.async_copy(
                rows_d.at[k], gd_ref.at[pl.ds(base0 + c * _CB, _CB)], sems_d[k])
            wb_e = pltpu.async_copy(
                rows_e.at[k], ep_ref.at[pl.ds(prow, _PB)], sems_e[k])
            wb_s.wait()
            wb_d.wait()
            wb_e.wait()

            @pl.when(c + _NSLOT < _NFULL)
            def _():
                start_chunk(c + _NSLOT, k)
        return carry

    lax.fori_loop(0, _NFULL // _NSLOT, body, 0)

    rbase = _NFULL * _CB
    rprow = pbase0 + _NFULL * _PB
    pltpu.async_copy(z_ref.at[idx_s.at[pl.ds(rbase, _REM)]],
                     rows_s.at[0, pl.ds(0, _REM)], sem_s0).wait()
    pltpu.async_copy(z_ref.at[idx_d.at[pl.ds(rbase, _REM)]],
                     rows_d.at[0, pl.ds(0, _REM)], sem_d0).wait()
    pltpu.async_copy(ea_ref.at[pl.ds(rprow, _PREM)],
                     rows_e.at[0, pl.ds(0, _PREM), :, pl.ds(0, _D_EDGE)],
                     sem_e0).wait()
    pltpu.async_copy(ea_ref.at[pl.ds(rprow, _PREM)],
                     rows_e.at[0, pl.ds(0, _PREM), :, pl.ds(_D_EDGE, _D_EDGE)],
                     sem_e0).wait()
    pltpu.sync_copy(rows_s.at[0, pl.ds(0, _REM)],
                    gs_ref.at[pl.ds(base0 + rbase, _REM)])
    pltpu.sync_copy(rows_d.at[0, pl.ds(0, _REM)],
                    gd_ref.at[pl.ds(base0 + rbase, _REM)])
    pltpu.sync_copy(rows_e.at[0, pl.ds(0, _PREM)],
                    ep_ref.at[pl.ds(rprow, _PREM)])


def _mlp_body(gsp_ref, gdp_ref, eap_ref, w1a_ref, w1b_ref, w1c_ref,
              b1r_ref, w2t_ref, b2_ref, out_ref):
    h = jnp.dot(gsp_ref[...], w1a_ref[...], preferred_element_type=jnp.float32)
    h = h + jnp.dot(gdp_ref[...], w1b_ref[...], preferred_element_type=jnp.float32)
    h = h + jnp.dot(eap_ref[...], w1c_ref[...], preferred_element_type=jnp.float32)
    h = jnp.maximum(h + b1r_ref[...], 0.0)
    logit = jax.lax.dot_general(
        w2t_ref[...], h, (((1,), (1,)), ((), ())),
        preferred_element_type=jnp.float32,
    )
    out_ref[...] = jax.nn.sigmoid(logit + b2_ref[0, 0])


def kernel(x, edge_index, edge_attr, W_ne, b_ne, W1, b1, W2, b2):
    bn = 1000
    z = pl.pallas_call(
        _encode_body,
        grid=(_N // bn,),
        in_specs=[
            pl.BlockSpec((bn, _D_FEAT), lambda i: (i, 0)),
            pl.BlockSpec((_D_FEAT, _HIDDEN), lambda i: (0, 0)),
            pl.BlockSpec((1, _HIDDEN), lambda i: (0, 0)),
        ],
        out_specs=pl.BlockSpec((bn, _HIDDEN), lambda i: (i, 0)),
        out_shape=jax.ShapeDtypeStruct((_N, _HIDDEN), jnp.float32),
    )(x, W_ne.T, b_ne.reshape(1, _HIDDEN))

    mesh = plsc.VectorSubcoreMesh(core_axis_name="c", subcore_axis_name="s")
    gs, gd, ep = pl.kernel(
        _sc_gather_body,
        out_type=(
            jax.ShapeDtypeStruct((_E, _HIDDEN), jnp.float32),
            jax.ShapeDtypeStruct((_E, _HIDDEN), jnp.float32),
            jax.ShapeDtypeStruct((_E // 4, 4, 2 * _D_EDGE), jnp.float32),
        ),
        mesh=mesh,
        compiler_params=pltpu.CompilerParams(use_tc_tiling_on_sc=False),
        scratch_types=[
            pltpu.VMEM((_EW,), jnp.int32),
            pltpu.VMEM((_EW,), jnp.int32),
            pltpu.VMEM((_NSLOT, _CB, _HIDDEN), jnp.float32),
            pltpu.VMEM((_NSLOT, _CB, _HIDDEN), jnp.float32),
            pltpu.VMEM((_NSLOT, _PB, 4, 2 * _D_EDGE), jnp.float32),
            pltpu.SemaphoreType.DMA,
            pltpu.SemaphoreType.DMA,
            pltpu.SemaphoreType.DMA,
            pltpu.SemaphoreType.DMA,
            pltpu.SemaphoreType.DMA,
            pltpu.SemaphoreType.DMA,
            pltpu.SemaphoreType.DMA,
            pltpu.SemaphoreType.DMA,
            pltpu.SemaphoreType.DMA,
        ],
    )(z, edge_index.reshape(2 * _E), edge_attr.reshape(_E // 4, 4, _D_EDGE))

    gsp = gs.reshape(_E // 4, 128)
    gdp = gd.reshape(_E // 4, 128)
    eap = ep.reshape(_E // 4, 128)

    eye4 = jnp.eye(4, dtype=jnp.float32)
    w1a = jnp.kron(eye4, W1[:, :_HIDDEN].T)
    w1b = jnp.kron(eye4, W1[:, _HIDDEN:2 * _HIDDEN].T)
    w1c = jnp.kron(eye4, jnp.concatenate(
        [W1[:, 2 * _HIDDEN:].T, jnp.zeros((_D_EDGE, _MLP_H), jnp.float32)],
        axis=0))
    w2t = jnp.kron(eye4, W2)
    b1r = jnp.tile(b1, 4).reshape(1, 4 * _MLP_H)

    out4 = pl.pallas_call(
        _mlp_body,
        grid=(_E // _BE,),
        in_specs=[
            pl.BlockSpec((_BP, 128), lambda i: (i, 0)),
            pl.BlockSpec((_BP, 128), lambda i: (i, 0)),
            pl.BlockSpec((_BP, 128), lambda i: (i, 0)),
            pl.BlockSpec((128, 256), lambda i: (0, 0)),
            pl.BlockSpec((128, 256), lambda i: (0, 0)),
            pl.BlockSpec((128, 256), lambda i: (0, 0)),
            pl.BlockSpec((1, 256), lambda i: (0, 0)),
            pl.BlockSpec((4, 256), lambda i: (0, 0)),
            pl.BlockSpec((1, 1), lambda i: (0, 0)),
        ],
        out_specs=pl.BlockSpec((4, _BP), lambda i: (0, i)),
        out_shape=jax.ShapeDtypeStruct((4, _E // 4), jnp.float32),
    )(
        gsp, gdp, eap, w1a, w1b, w1c, b1r, w2t, b2.reshape(1, 1),
    )
    return out4.T.reshape(_E)


# speedup vs baseline: 1.9820x; 1.9820x over previous
import jax
import jax.numpy as jnp
from jax import lax
from jax.experimental import pallas as pl
from jax.experimental.pallas import tpu as pltpu
from jax.experimental.pallas import tpu_sc as plsc

_N = 10000
_E = 320000
_D_FEAT = 128
_HIDDEN = 32
_MLP_H = 64
_D_EDGE = 16

_NC = 2
_NS = 16
_NW = _NC * _NS
_EW = _E // _NW
_CB = 128
_NFULL = _EW // _CB
_REM = _EW - _NFULL * _CB
_NSLOT = 3

_BE = 12800
_BP = _BE // 4


def _encode_body(x_ref, wnet_ref, bne_ref, z_ref):
    z_ref[...] = (
        jnp.dot(x_ref[...], wnet_ref[...], preferred_element_type=jnp.float32)
        + bne_ref[...]
    )


def _sc_gather_body(z_ref, ei_ref, gs_ref, gd_ref,
                    idx_s, idx_d, rows_s, rows_d,
                    sem_s0, sem_s1, sem_s2, sem_d0, sem_d1, sem_d2):
    wid = lax.axis_index("s") * _NC + lax.axis_index("c")
    base0 = wid * _EW
    sems_s = (sem_s0, sem_s1, sem_s2)
    sems_d = (sem_d0, sem_d1, sem_d2)

    pltpu.sync_copy(ei_ref.at[0, pl.ds(base0, _EW)], idx_s)
    pltpu.sync_copy(ei_ref.at[1, pl.ds(base0, _EW)], idx_d)

    for k in range(_NSLOT):
        pltpu.async_copy(z_ref.at[idx_s.at[pl.ds(k * _CB, _CB)]],
                         rows_s.at[k], sems_s[k])
        pltpu.async_copy(z_ref.at[idx_d.at[pl.ds(k * _CB, _CB)]],
                         rows_d.at[k], sems_d[k])

    def body(i, carry):
        for k in range(_NSLOT):
            c = i * _NSLOT + k
            pltpu.make_async_copy(z_ref.at[idx_s.at[pl.ds(c * _CB, _CB)]],
                                  rows_s.at[k], sems_s[k]).wait()
            pltpu.make_async_copy(z_ref.at[idx_d.at[pl.ds(c * _CB, _CB)]],
                                  rows_d.at[k], sems_d[k]).wait()
            wb_s = pltpu.async_copy(
                rows_s.at[k], gs_ref.at[pl.ds(base0 + c * _CB, _CB)], sems_s[k])
            wb_d = pltpu.async_copy(
                rows_d.at[k], gd_ref.at[pl.ds(base0 + c * _CB, _CB)], sems_d[k])
            wb_s.wait()
            wb_d.wait()

            @pl.when(c + _NSLOT < _NFULL)
            def _():
                nc = c + _NSLOT
                pltpu.async_copy(z_ref.at[idx_s.at[pl.ds(nc * _CB, _CB)]],
                                 rows_s.at[k], sems_s[k])
                pltpu.async_copy(z_ref.at[idx_d.at[pl.ds(nc * _CB, _CB)]],
                                 rows_d.at[k], sems_d[k])
        return carry

    lax.fori_loop(0, _NFULL // _NSLOT, body, 0)

    rbase = _NFULL * _CB
    pltpu.async_copy(z_ref.at[idx_s.at[pl.ds(rbase, _REM)]],
                     rows_s.at[0, pl.ds(0, _REM)], sem_s0).wait()
    pltpu.async_copy(z_ref.at[idx_d.at[pl.ds(rbase, _REM)]],
                     rows_d.at[0, pl.ds(0, _REM)], sem_d0).wait()
    pltpu.sync_copy(rows_s.at[0, pl.ds(0, _REM)],
                    gs_ref.at[pl.ds(base0 + rbase, _REM)])
    pltpu.sync_copy(rows_d.at[0, pl.ds(0, _REM)],
                    gd_ref.at[pl.ds(base0 + rbase, _REM)])


def _mlp_body(gsp_ref, gdp_ref, eap_ref, w1a_ref, w1b_ref, w1c_ref,
              b1r_ref, w2t_ref, b2_ref, out_ref):
    h = jnp.dot(gsp_ref[...], w1a_ref[...], preferred_element_type=jnp.float32)
    h = h + jnp.dot(gdp_ref[...], w1b_ref[...], preferred_element_type=jnp.float32)
    h = h + jnp.dot(eap_ref[...], w1c_ref[...], preferred_element_type=jnp.float32)
    h = jnp.maximum(h + b1r_ref[...], 0.0)
    logit = jax.lax.dot_general(
        w2t_ref[...], h, (((1,), (1,)), ((), ())),
        preferred_element_type=jnp.float32,
    )
    out_ref[...] = jax.nn.sigmoid(logit + b2_ref[0, 0])


def kernel(x, edge_index, edge_attr, W_ne, b_ne, W1, b1, W2, b2):
    bn = 1000
    z = pl.pallas_call(
        _encode_body,
        grid=(_N // bn,),
        in_specs=[
            pl.BlockSpec((bn, _D_FEAT), lambda i: (i, 0)),
            pl.BlockSpec((_D_FEAT, _HIDDEN), lambda i: (0, 0)),
            pl.BlockSpec((1, _HIDDEN), lambda i: (0, 0)),
        ],
        out_specs=pl.BlockSpec((bn, _HIDDEN), lambda i: (i, 0)),
        out_shape=jax.ShapeDtypeStruct((_N, _HIDDEN), jnp.float32),
    )(x, W_ne.T, b_ne.reshape(1, _HIDDEN))

    mesh = plsc.VectorSubcoreMesh(core_axis_name="c", subcore_axis_name="s")
    gs, gd = pl.kernel(
        _sc_gather_body,
        out_type=(
            jax.ShapeDtypeStruct((_E, _HIDDEN), jnp.float32),
            jax.ShapeDtypeStruct((_E, _HIDDEN), jnp.float32),
        ),
        mesh=mesh,
        compiler_params=pltpu.CompilerParams(use_tc_tiling_on_sc=False),
        scratch_types=[
            pltpu.VMEM((_EW,), jnp.int32),
            pltpu.VMEM((_EW,), jnp.int32),
            pltpu.VMEM((_NSLOT, _CB, _HIDDEN), jnp.float32),
            pltpu.VMEM((_NSLOT, _CB, _HIDDEN), jnp.float32),
            pltpu.SemaphoreType.DMA,
            pltpu.SemaphoreType.DMA,
            pltpu.SemaphoreType.DMA,
            pltpu.SemaphoreType.DMA,
            pltpu.SemaphoreType.DMA,
            pltpu.SemaphoreType.DMA,
        ],
    )(z, edge_index)

    gsp = gs.reshape(_E // 4, 128)
    gdp = gd.reshape(_E // 4, 128)
    eap = edge_attr.reshape(_E // 4, 4 * _D_EDGE)

    eye4 = jnp.eye(4, dtype=jnp.float32)
    w1a = jnp.kron(eye4, W1[:, :_HIDDEN].T)
    w1b = jnp.kron(eye4, W1[:, _HIDDEN:2 * _HIDDEN].T)
    w1c = jnp.kron(eye4, W1[:, 2 * _HIDDEN:].T)
    w2t = jnp.kron(eye4, W2)
    b1r = jnp.tile(b1, 4).reshape(1, 4 * _MLP_H)

    out4 = pl.pallas_call(
        _mlp_body,
        grid=(_E // _BE,),
        in_specs=[
            pl.BlockSpec((_BP, 128), lambda i: (i, 0)),
            pl.BlockSpec((_BP, 128), lambda i: (i, 0)),
            pl.BlockSpec((_BP, 4 * _D_EDGE), lambda i: (i, 0)),
            pl.BlockSpec((128, 256), lambda i: (0, 0)),
            pl.BlockSpec((128, 256), lambda i: (0, 0)),
            pl.BlockSpec((4 * _D_EDGE, 256), lambda i: (0, 0)),
            pl.BlockSpec((1, 256), lambda i: (0, 0)),
            pl.BlockSpec((4, 256), lambda i: (0, 0)),
            pl.BlockSpec((1, 1), lambda i: (0, 0)),
        ],
        out_specs=pl.BlockSpec((4, _BP), lambda i: (0, i)),
        out_shape=jax.ShapeDtypeStruct((4, _E // 4), jnp.float32),
    )(
        gsp, gdp, eap, w1a, w1b, w1c, b1r, w2t, b2.reshape(1, 1),
    )
    return out4.T.reshape(_E)
